# baseline (speedup 1.0000x reference)
import functools

import jax
import jax.numpy as jnp
from jax import lax
from jax.experimental import pallas as pl
from jax.experimental.pallas import tpu as pltpu
from jax.experimental.pallas import tpu_sc as plsc

N = 10000
E = 320000
HF = 64
NH = 3
DH = NH * HF
NCLS = 40
NLAYERS = 8

NCORES = 2
NSUB = 16
NTILES = NCORES * NSUB
EPT = E // NTILES
CH = 80
NCHUNK = EPT // CH
NB = 5
ROWP = 80
NPT = N // NSUB
SED = 48

BN = 2048
GRID = (N + BN - 1) // BN


def _split_outs(z, z0_ref, z1_ref, z2_ref, esedT_ref, act):
    z0_ref[...] = z[:, 0:HF]
    z1_ref[...] = z[:, HF:2 * HF]
    z2_ref[...] = z[:, 2 * HF:3 * HF]
    esedT_ref[...] = lax.dot_general(
        act, z, (((1,), (1,)), ((), ())), preferred_element_type=jnp.float32)


def _l1_body(x_ref, wc_ref, act_ref, z0_ref, z1_ref, z2_ref, esedT_ref):
    z = jnp.dot(x_ref[...], wc_ref[...], preferred_element_type=jnp.float32)
    _split_outs(z, z0_ref, z1_ref, z2_ref, esedT_ref, act_ref[...])


def _elu_heads(acc_refs, ad_ref):
    hks = []
    for k, a_ref in enumerate(acc_refs):
        num = a_ref[0] + a_ref[1]
        den = ad_ref[k, 0, :, 0:1] + ad_ref[k, 1, :, 0:1]
        den = jnp.where(den != 0.0, den, 1.0)
        hk = num / den
        hk = jnp.where(hk > 0.0, hk, jnp.exp(hk) - 1.0)
        hks.append(hk)
    return hks


def _lmid_body(a0_ref, a1_ref, a2_ref, ad_ref, wc3_ref, act_ref,
               z0_ref, z1_ref, z2_ref, esedT_ref):
    hks = _elu_heads((a0_ref, a1_ref, a2_ref), ad_ref)
    z = sum(jnp.dot(hks[k], wc3_ref[k], preferred_element_type=jnp.float32)
            for k in range(NH))
    _split_outs(z, z0_ref, z1_ref, z2_ref, esedT_ref, act_ref[...])


def _final_body(a0_ref, a1_ref, a2_ref, ad_ref, fcw3_ref, fcb_ref, out_ref):
    hks = _elu_heads((a0_ref, a1_ref, a2_ref), ad_ref)
    logits = sum(jnp.dot(hks[k], fcw3_ref[k], preferred_element_type=jnp.float32)
                 for k in range(NH))
    out_ref[...] = logits + fcb_ref[...]


_ZOUT = (
    [jax.ShapeDtypeStruct((N, HF), jnp.float32)] * NH
    + [jax.ShapeDtypeStruct((SED, N), jnp.float32)]
)
_ZSPECS = (
    [pl.BlockSpec((BN, HF), lambda i: (i, 0))] * NH
    + [pl.BlockSpec((SED, BN), lambda i: (0, i))]
)
_ACCSPEC = pl.BlockSpec((NCORES, BN, HF), lambda i: (0, i, 0))
_ADSPEC = pl.BlockSpec((NH, NCORES, BN, 16), lambda i: (0, 0, i, 0))


def _tc_layer1(x, wc, act):
    return pl.pallas_call(
        _l1_body,
        grid=(GRID,),
        in_specs=[
            pl.BlockSpec((BN, 128), lambda i: (i, 0)),
            pl.BlockSpec((128, DH), lambda i: (0, 0)),
            pl.BlockSpec((SED, DH), lambda i: (0, 0)),
        ],
        out_specs=_ZSPECS,
        out_shape=_ZOUT,
    )(x, wc, act)


def _tc_layermid(a0, a1, a2, ad, wc3, act):
    return pl.pallas_call(
        _lmid_body,
        grid=(GRID,),
        in_specs=[
            _ACCSPEC, _ACCSPEC, _ACCSPEC, _ADSPEC,
            pl.BlockSpec((NH, HF, DH), lambda i: (0, 0, 0)),
            pl.BlockSpec((SED, DH), lambda i: (0, 0)),
        ],
        out_specs=_ZSPECS,
        out_shape=_ZOUT,
    )(a0, a1, a2, ad, wc3, act)


def _tc_final(a0, a1, a2, ad, fcw3, fcb):
    return pl.pallas_call(
        _final_body,
        grid=(GRID,),
        in_specs=[
            _ACCSPEC, _ACCSPEC, _ACCSPEC, _ADSPEC,
            pl.BlockSpec((NH, HF, NCLS), lambda i: (0, 0, 0)),
            pl.BlockSpec((1, NCLS), lambda i: (0, 0)),
        ],
        out_specs=pl.BlockSpec((BN, NCLS), lambda i: (i, 0)),
        out_shape=jax.ShapeDtypeStruct((N, NCLS), jnp.float32),
    )(a0, a1, a2, ad, fcw3, fcb)


_SC_OUT = (
    tuple(jax.ShapeDtypeStruct((NCORES * N, HF), jnp.float32)
          for _ in range(NH))
    + (jax.ShapeDtypeStruct((NH, NCORES * N, 16), jnp.float32),)
)

_SC_SCRATCH = [
    pltpu.VMEM((N,), jnp.float32),
    pltpu.VMEM((N,), jnp.float32),
    pltpu.VMEM((NCHUNK, CH), jnp.int32),
    pltpu.VMEM((NCHUNK, CH), jnp.int32),
    pltpu.VMEM((NB, CH, HF), jnp.float32),
    pltpu.VMEM((NB, CH, 16), jnp.float32),
    pltpu.VMEM((NB * CH,), jnp.float32),
    pltpu.VMEM((16,), jnp.float32),
    pltpu.VMEM_SHARED((N, HF), jnp.float32),
    pltpu.VMEM_SHARED((N, 16), jnp.float32),
] + [pltpu.SemaphoreType.DMA] * (3 * NB)


def _sc_body(src2_hbm, dst2_hbm, z0_hbm, z1_hbm, z2_hbm, esedT_hbm,
             zr_hbm, zd_hbm,
             o0_hbm, o1_hbm, o2_hbm, od_hbm,
             es_v, ed_v, srcv, dstv, rowsv, exrow, exv, mbuf,
             rowacc, denacc, *sems):
    gsems = sems[:NB]
    ssems = sems[NB:2 * NB]
    dsems = sems[2 * NB:]
    cid = lax.axis_index("c")
    sid = lax.axis_index("s")
    zvec = jnp.zeros((16,), jnp.float32)
    lane0 = jnp.where(lax.iota(jnp.int32, 16) == 0, 1.0, 0.0)

    trow0 = (cid * NSUB + sid) * NCHUNK
    pltpu.sync_copy(src2_hbm.at[pl.ds(trow0, NCHUNK)], srcv)
    pltpu.sync_copy(dst2_hbm.at[pl.ds(trow0, NCHUNK)], dstv)

    z_hbms = (z0_hbm, z1_hbm, z2_hbm)
    o_hbms = (o0_hbm, o1_hbm, o2_hbm)
    r0 = sid * NPT
    for h in range(NH):
        pltpu.sync_copy(zr_hbm.at[pl.ds(r0, NPT)], rowacc.at[pl.ds(r0, NPT)])
        pltpu.sync_copy(zd_hbm.at[pl.ds(r0, NPT)], denacc.at[pl.ds(r0, NPT)])
        pltpu.sync_copy(esedT_hbm.at[8 * h], es_v)
        pltpu.sync_copy(esedT_hbm.at[8 * (NH + h)], ed_v)
        plsc.subcore_barrier()

        def _mx(i, m):
            return jnp.maximum(m, es_v[pl.ds(i * 16, 16)])
        mvec = lax.fori_loop(0, N // 16, _mx,
                             jnp.full((16,), -1e30, jnp.float32))
        mbuf[pl.ds(0, 16)] = mvec
        maxes = plsc.load_gather(mbuf, [jnp.zeros((16,), jnp.int32)])
        for i in range(1, 16):
            maxes = jnp.maximum(
                maxes,
                plsc.load_gather(mbuf, [jnp.full((16,), i, jnp.int32)]))

        for b in range(NB):
            pltpu.async_copy(
                z_hbms[h].at[srcv.at[b]], rowsv.at[b], gsems[b])

        def _group(g, _):
            for b in range(NB):
                k = g * NB + b

                def _sc16(i, _, k=k, b=b):
                    si = srcv[k, pl.ds(i * 16, 16)]
                    di = dstv[k, pl.ds(i * 16, 16)]
                    esg = plsc.load_gather(es_v, [si])
                    edg = plsc.load_gather(ed_v, [di])
                    e = esg + edg
                    e = jnp.where(e >= 0.0, e, 0.01 * e)
                    m0 = edg + maxes
                    m0 = jnp.where(m0 >= 0.0, m0, 0.01 * m0)
                    exv[pl.ds(b * CH + i * 16, 16)] = jnp.exp(
                        jnp.maximum(e - m0, -75.0))
                    return 0
                lax.fori_loop(0, CH // 16, _sc16, 0)
            scps = []
            for b in range(NB):
                k = g * NB + b
                pltpu.make_async_copy(
                    z_hbms[h].at[srcv.at[k]], rowsv.at[b], gsems[b]).wait()

                def _row(r4, _, b=b):
                    for u in range(4):
                        r = r4 * 4 + u
                        exs = plsc.load_gather(
                            exv, [jnp.full((16,), b * CH, jnp.int32) + r])
                        for j in range(HF // 16):
                            rowsv[b, r, pl.ds(j * 16, 16)] = (
                                rowsv[b, r, pl.ds(j * 16, 16)] * exs)
                        exrow[b, r, pl.ds(0, 16)] = exs * lane0
                    return 0
                lax.fori_loop(0, CH // 4, _row, 0)
                scps.append(pltpu.async_copy(
                    rowsv.at[b], rowacc.at[dstv.at[k]], ssems[b], add=True))
                scps.append(pltpu.async_copy(
                    exrow.at[b], denacc.at[dstv.at[k]], dsems[b], add=True))
            for b in range(NB):
                scps[2 * b].wait()
                scps[2 * b + 1].wait()

                @pl.when(g + 1 < NCHUNK // NB)
                def _(b=b, g=g):
                    k = (g + 1) * NB + b
                    pltpu.async_copy(
                        z_hbms[h].at[srcv.at[k]], rowsv.at[b], gsems[b])
            return 0
        lax.fori_loop(0, NCHUNK // NB, _group, 0)
        plsc.subcore_barrier()

        pltpu.sync_copy(rowacc.at[pl.ds(r0, NPT)],
                        o_hbms[h].at[pl.ds(cid * N + r0, NPT)])
        pltpu.sync_copy(denacc.at[pl.ds(r0, NPT)],
                        od_hbm.at[h].at[pl.ds(cid * N + r0, NPT)])
        plsc.subcore_barrier()


@functools.cache
def _get_sc_agg():
    mesh = plsc.VectorSubcoreMesh(
        core_axis_name="c", subcore_axis_name="s",
        num_cores=NCORES, num_subcores=NSUB)
    return pl.kernel(
        _sc_body,
        out_type=_SC_OUT,
        mesh=mesh,
        compiler_params=pltpu.CompilerParams(
            needs_layout_passes=False, use_tc_tiling_on_sc=False),
        scratch_types=_SC_SCRATCH,
    )


def _sc_agg(*args):
    return _get_sc_agg()(*args)


def _make_act(A):
    act = jnp.zeros((SED, DH), jnp.float32)
    for h in range(NH):
        act = act.at[8 * h, h * HF:(h + 1) * HF].set(A[h, :HF, 0])
        act = act.at[8 * (NH + h), h * HF:(h + 1) * HF].set(A[h, HF:, 0])
    return act


def kernel(x, edge_index, W1, A1, Wrest, Arest, fc_w, fc_b):
    src_s, dst_s = lax.sort_key_val(edge_index[0], edge_index[1])
    src = src_s.reshape(E // CH, CH)
    dst = dst_s.reshape(E // CH, CH)

    wc1 = jnp.concatenate([W1[h] for h in range(NH)], axis=1)
    act1 = _make_act(A1)
    wc3s = [jnp.transpose(Wrest[l], (1, 0, 2)).reshape(NH, HF, DH)
            for l in range(NLAYERS - 1)]
    acts = [_make_act(Arest[l]) for l in range(NLAYERS - 1)]
    fcw3 = fc_w.reshape(NH, HF, NCLS)
    fcb = fc_b.reshape(1, NCLS)

    zr = jnp.zeros((N, HF), jnp.float32)
    zd = jnp.zeros((N, 16), jnp.float32)

    z0, z1, z2, esedT = _tc_layer1(x, wc1, act1)
    for l in range(NLAYERS - 1):
        a0, a1, a2, ad = _sc_agg(src, dst, z0, z1, z2, esedT, zr, zd)
        a0 = a0.reshape(NCORES, N, HF)
        a1 = a1.reshape(NCORES, N, HF)
        a2 = a2.reshape(NCORES, N, HF)
        ad = ad.reshape(NH, NCORES, N, 16)
        z0, z1, z2, esedT = _tc_layermid(a0, a1, a2, ad, wc3s[l], acts[l])
    a0, a1, a2, ad = _sc_agg(src, dst, z0, z1, z2, esedT, zr, zd)
    a0 = a0.reshape(NCORES, N, HF)
    a1 = a1.reshape(NCORES, N, HF)
    a2 = a2.reshape(NCORES, N, HF)
    ad = ad.reshape(NH, NCORES, N, 16)
    return _tc_final(a0, a1, a2, ad, fcw3, fcb)


# speedup vs baseline: 1.5957x; 1.5957x over previous
import functools

import jax
import jax.numpy as jnp
from jax import lax
from jax.experimental import pallas as pl
from jax.experimental.pallas import tpu as pltpu
from jax.experimental.pallas import tpu_sc as plsc

N = 10000
E = 320000
HF = 64
NH = 3
DH = NH * HF
NCLS = 40
NLAYERS = 8

NCORES = 2
NSUB = 16
NTILES = NCORES * NSUB
EPT = E // NTILES
CH = 80
NCHUNK = EPT // CH
NB = 5
ROWP = 80
NPT = N // NSUB
SED = 48

BN = 2048
GRID = (N + BN - 1) // BN


def _split_outs(z, z0_ref, z1_ref, z2_ref, esedT_ref, act):
    z0_ref[...] = z[:, 0:HF]
    z1_ref[...] = z[:, HF:2 * HF]
    z2_ref[...] = z[:, 2 * HF:3 * HF]
    esedT_ref[...] = lax.dot_general(
        act, z, (((1,), (1,)), ((), ())), preferred_element_type=jnp.float32)


def _l1_body(x_ref, wc_ref, act_ref, z0_ref, z1_ref, z2_ref, esedT_ref):
    z = jnp.dot(x_ref[...], wc_ref[...], preferred_element_type=jnp.float32)
    _split_outs(z, z0_ref, z1_ref, z2_ref, esedT_ref, act_ref[...])


def _elu_heads(acc_refs, ad_ref):
    hks = []
    for k, a_ref in enumerate(acc_refs):
        num = a_ref[0] + a_ref[1]
        den = ad_ref[k, 0, :, 0:1] + ad_ref[k, 1, :, 0:1]
        den = jnp.where(den != 0.0, den, 1.0)
        hk = num / den
        hk = jnp.where(hk > 0.0, hk, jnp.exp(hk) - 1.0)
        hks.append(hk)
    return hks


def _lmid_body(a0_ref, a1_ref, a2_ref, ad_ref, wc3_ref, act_ref,
               z0_ref, z1_ref, z2_ref, esedT_ref):
    hks = _elu_heads((a0_ref, a1_ref, a2_ref), ad_ref)
    z = sum(jnp.dot(hks[k], wc3_ref[k], preferred_element_type=jnp.float32)
            for k in range(NH))
    _split_outs(z, z0_ref, z1_ref, z2_ref, esedT_ref, act_ref[...])


def _final_body(a0_ref, a1_ref, a2_ref, ad_ref, fcw3_ref, fcb_ref, out_ref):
    hks = _elu_heads((a0_ref, a1_ref, a2_ref), ad_ref)
    logits = sum(jnp.dot(hks[k], fcw3_ref[k], preferred_element_type=jnp.float32)
                 for k in range(NH))
    out_ref[...] = logits + fcb_ref[...]


_ZOUT = (
    [jax.ShapeDtypeStruct((N, HF), jnp.float32)] * NH
    + [jax.ShapeDtypeStruct((SED, N), jnp.float32)]
)
_ZSPECS = (
    [pl.BlockSpec((BN, HF), lambda i: (i, 0))] * NH
    + [pl.BlockSpec((SED, BN), lambda i: (0, i))]
)
_ACCSPEC = pl.BlockSpec((NCORES, BN, HF), lambda i: (0, i, 0))
_ADSPEC = pl.BlockSpec((NH, NCORES, BN, 16), lambda i: (0, 0, i, 0))


def _tc_layer1(x, wc, act):
    return pl.pallas_call(
        _l1_body,
        grid=(GRID,),
        in_specs=[
            pl.BlockSpec((BN, 128), lambda i: (i, 0)),
            pl.BlockSpec((128, DH), lambda i: (0, 0)),
            pl.BlockSpec((SED, DH), lambda i: (0, 0)),
        ],
        out_specs=_ZSPECS,
        out_shape=_ZOUT,
    )(x, wc, act)


def _tc_layermid(a0, a1, a2, ad, wc3, act):
    return pl.pallas_call(
        _lmid_body,
        grid=(GRID,),
        in_specs=[
            _ACCSPEC, _ACCSPEC, _ACCSPEC, _ADSPEC,
            pl.BlockSpec((NH, HF, DH), lambda i: (0, 0, 0)),
            pl.BlockSpec((SED, DH), lambda i: (0, 0)),
        ],
        out_specs=_ZSPECS,
        out_shape=_ZOUT,
    )(a0, a1, a2, ad, wc3, act)


def _tc_final(a0, a1, a2, ad, fcw3, fcb):
    return pl.pallas_call(
        _final_body,
        grid=(GRID,),
        in_specs=[
            _ACCSPEC, _ACCSPEC, _ACCSPEC, _ADSPEC,
            pl.BlockSpec((NH, HF, NCLS), lambda i: (0, 0, 0)),
            pl.BlockSpec((1, NCLS), lambda i: (0, 0)),
        ],
        out_specs=pl.BlockSpec((BN, NCLS), lambda i: (i, 0)),
        out_shape=jax.ShapeDtypeStruct((N, NCLS), jnp.float32),
    )(a0, a1, a2, ad, fcw3, fcb)


_SC_OUT = (
    tuple(jax.ShapeDtypeStruct((NCORES * N, HF), jnp.float32)
          for _ in range(NH))
    + (jax.ShapeDtypeStruct((NH, NCORES * N, 16), jnp.float32),)
)

_SC_SCRATCH = [
    pltpu.VMEM((N,), jnp.float32),
    pltpu.VMEM((N,), jnp.float32),
    pltpu.VMEM((NCHUNK, CH), jnp.int32),
    pltpu.VMEM((NCHUNK, CH), jnp.int32),
    pltpu.VMEM((NB, CH, HF), jnp.float32),
    pltpu.VMEM((NB, CH, 16), jnp.float32),
    pltpu.VMEM((NB * CH,), jnp.float32),
    pltpu.VMEM((16,), jnp.float32),
    pltpu.VMEM_SHARED((N, HF), jnp.float32),
    pltpu.VMEM_SHARED((N, 16), jnp.float32),
] + [pltpu.SemaphoreType.DMA] * (3 * NB)


def _sc_body(src2_hbm, dst2_hbm, z0_hbm, z1_hbm, z2_hbm, esedT_hbm,
             zr_hbm, zd_hbm,
             o0_hbm, o1_hbm, o2_hbm, od_hbm,
             es_v, ed_v, srcv, dstv, rowsv, exrow, exv, mbuf,
             rowacc, denacc, *sems):
    gsems = sems[:NB]
    ssems = sems[NB:2 * NB]
    dsems = sems[2 * NB:]
    cid = lax.axis_index("c")
    sid = lax.axis_index("s")
    zvec = jnp.zeros((16,), jnp.float32)
    lane0 = jnp.where(lax.iota(jnp.int32, 16) == 0, 1.0, 0.0)

    trow0 = (cid * NSUB + sid) * NCHUNK
    pltpu.sync_copy(src2_hbm.at[pl.ds(trow0, NCHUNK)], srcv)
    pltpu.sync_copy(dst2_hbm.at[pl.ds(trow0, NCHUNK)], dstv)

    z_hbms = (z0_hbm, z1_hbm, z2_hbm)
    o_hbms = (o0_hbm, o1_hbm, o2_hbm)
    r0 = sid * NPT
    for h in range(NH):
        pltpu.sync_copy(zr_hbm.at[pl.ds(r0, NPT)], rowacc.at[pl.ds(r0, NPT)])
        pltpu.sync_copy(zd_hbm.at[pl.ds(r0, NPT)], denacc.at[pl.ds(r0, NPT)])
        pltpu.sync_copy(esedT_hbm.at[8 * h], es_v)
        pltpu.sync_copy(esedT_hbm.at[8 * (NH + h)], ed_v)
        plsc.subcore_barrier()

        def _mx(i, m):
            return jnp.maximum(m, es_v[pl.ds(i * 16, 16)])
        mvec = lax.fori_loop(0, N // 16, _mx,
                             jnp.full((16,), -1e30, jnp.float32))
        mbuf[pl.ds(0, 16)] = mvec
        maxes = plsc.load_gather(mbuf, [jnp.zeros((16,), jnp.int32)])
        for i in range(1, 16):
            maxes = jnp.maximum(
                maxes,
                plsc.load_gather(mbuf, [jnp.full((16,), i, jnp.int32)]))

        for b in range(NB):
            pltpu.async_copy(
                z_hbms[h].at[srcv.at[b]], rowsv.at[b], gsems[b])

        def _group(g, _):
            for b in range(NB):
                k = g * NB + b

                def _sc16(i, _, k=k, b=b):
                    si = srcv[k, pl.ds(i * 16, 16)]
                    di = dstv[k, pl.ds(i * 16, 16)]
                    esg = plsc.load_gather(es_v, [si])
                    edg = plsc.load_gather(ed_v, [di])
                    e = esg + edg
                    e = jnp.where(e >= 0.0, e, 0.01 * e)
                    m0 = edg + maxes
                    m0 = jnp.where(m0 >= 0.0, m0, 0.01 * m0)
                    exv[pl.ds(b * CH + i * 16, 16)] = jnp.exp(
                        jnp.maximum(e - m0, -75.0))
                    return 0
                lax.fori_loop(0, CH // 16, _sc16, 0)
            scps = []
            for b in range(NB):
                k = g * NB + b
                pltpu.make_async_copy(
                    z_hbms[h].at[srcv.at[k]], rowsv.at[b], gsems[b]).wait()

                def _row(r4, _, b=b):
                    for u in range(4):
                        r = r4 * 4 + u
                        exs = plsc.load_gather(
                            exv, [jnp.full((16,), b * CH, jnp.int32) + r])
                        for j in range(HF // 16):
                            rowsv[b, r, pl.ds(j * 16, 16)] = (
                                rowsv[b, r, pl.ds(j * 16, 16)] * exs)
                        exrow[b, r, pl.ds(0, 16)] = exs * lane0
                    return 0
                lax.fori_loop(0, CH // 4, _row, 0)
                scps.append(pltpu.async_copy(
                    rowsv.at[b], rowacc.at[dstv.at[k]], ssems[b], add=True))
                scps.append(pltpu.async_copy(
                    exrow.at[b], denacc.at[dstv.at[k]], dsems[b], add=True))
            for b in range(NB):
                scps[2 * b].wait()
                scps[2 * b + 1].wait()

                @pl.when(g + 1 < NCHUNK // NB)
                def _(b=b, g=g):
                    k = (g + 1) * NB + b
                    pltpu.async_copy(
                        z_hbms[h].at[srcv.at[k]], rowsv.at[b], gsems[b])
            return 0
        lax.fori_loop(0, NCHUNK // NB, _group, 0)
        plsc.subcore_barrier()

        pltpu.sync_copy(rowacc.at[pl.ds(r0, NPT)],
                        o_hbms[h].at[pl.ds(cid * N + r0, NPT)])
        pltpu.sync_copy(denacc.at[pl.ds(r0, NPT)],
                        od_hbm.at[h].at[pl.ds(cid * N + r0, NPT)])
        plsc.subcore_barrier()


@functools.cache
def _get_sc_agg():
    mesh = plsc.VectorSubcoreMesh(
        core_axis_name="c", subcore_axis_name="s",
        num_cores=NCORES, num_subcores=NSUB)
    return pl.kernel(
        _sc_body,
        out_type=_SC_OUT,
        mesh=mesh,
        compiler_params=pltpu.CompilerParams(
            needs_layout_passes=False, use_tc_tiling_on_sc=False),
        scratch_types=_SC_SCRATCH,
    )


def _sc_agg(*args):
    return _get_sc_agg()(*args)


def _make_act(A):
    act = jnp.zeros((SED, DH), jnp.float32)
    for h in range(NH):
        act = act.at[8 * h, h * HF:(h + 1) * HF].set(A[h, :HF, 0])
        act = act.at[8 * (NH + h), h * HF:(h + 1) * HF].set(A[h, HF:, 0])
    return act


def kernel(x, edge_index, W1, A1, Wrest, Arest, fc_w, fc_b):
    src = edge_index[0].reshape(E // CH, CH)
    dst = edge_index[1].reshape(E // CH, CH)

    wc1 = jnp.concatenate([W1[h] for h in range(NH)], axis=1)
    act1 = _make_act(A1)
    wc3s = [jnp.transpose(Wrest[l], (1, 0, 2)).reshape(NH, HF, DH)
            for l in range(NLAYERS - 1)]
    acts = [_make_act(Arest[l]) for l in range(NLAYERS - 1)]
    fcw3 = fc_w.reshape(NH, HF, NCLS)
    fcb = fc_b.reshape(1, NCLS)

    zr = jnp.zeros((N, HF), jnp.float32)
    zd = jnp.zeros((N, 16), jnp.float32)

    z0, z1, z2, esedT = _tc_layer1(x, wc1, act1)
    for l in range(NLAYERS - 1):
        a0, a1, a2, ad = _sc_agg(src, dst, z0, z1, z2, esedT, zr, zd)
        a0 = a0.reshape(NCORES, N, HF)
        a1 = a1.reshape(NCORES, N, HF)
        a2 = a2.reshape(NCORES, N, HF)
        ad = ad.reshape(NH, NCORES, N, 16)
        z0, z1, z2, esedT = _tc_layermid(a0, a1, a2, ad, wc3s[l], acts[l])
    a0, a1, a2, ad = _sc_agg(src, dst, z0, z1, z2, esedT, zr, zd)
    a0 = a0.reshape(NCORES, N, HF)
    a1 = a1.reshape(NCORES, N, HF)
    a2 = a2.reshape(NCORES, N, HF)
    ad = ad.reshape(NH, NCORES, N, 16)
    return _tc_final(a0, a1, a2, ad, fcw3, fcb)


# speedup vs baseline: 1.7032x; 1.0674x over previous
import functools

import jax
import jax.numpy as jnp
from jax import lax
from jax.experimental import pallas as pl
from jax.experimental.pallas import tpu as pltpu
from jax.experimental.pallas import tpu_sc as plsc

N = 10000
E = 320000
HF = 64
NH = 3
DH = NH * HF
NCLS = 40
NLAYERS = 8

NCORES = 2
NSUB = 16
NTILES = NCORES * NSUB
EPT = E // NTILES
CH = 80
NCHUNK = EPT // CH
NB = 5
ROWP = 80
NPT = N // NSUB
SED = 48

BN = 2048
GRID = (N + BN - 1) // BN


def _split_outs(z, z0_ref, z1_ref, z2_ref, esedT_ref, act):
    z0_ref[...] = z[:, 0:HF]
    z1_ref[...] = z[:, HF:2 * HF]
    z2_ref[...] = z[:, 2 * HF:3 * HF]
    esedT_ref[...] = lax.dot_general(
        act, z, (((1,), (1,)), ((), ())), preferred_element_type=jnp.float32)


def _l1_body(x_ref, wc_ref, act_ref, z0_ref, z1_ref, z2_ref, esedT_ref):
    z = jnp.dot(x_ref[...], wc_ref[...], preferred_element_type=jnp.float32)
    _split_outs(z, z0_ref, z1_ref, z2_ref, esedT_ref, act_ref[...])


def _elu_heads(acc_refs, ad_ref):
    hks = []
    for k, a_ref in enumerate(acc_refs):
        num = a_ref[0] + a_ref[1]
        den = ad_ref[k, 0, :, 0:1] + ad_ref[k, 1, :, 0:1]
        den = jnp.where(den != 0.0, den, 1.0)
        hk = num / den
        hk = jnp.where(hk > 0.0, hk, jnp.exp(hk) - 1.0)
        hks.append(hk)
    return hks


def _lmid_body(a0_ref, a1_ref, a2_ref, ad_ref, wc3_ref, act_ref,
               z0_ref, z1_ref, z2_ref, esedT_ref):
    hks = _elu_heads((a0_ref, a1_ref, a2_ref), ad_ref)
    z = sum(jnp.dot(hks[k], wc3_ref[k], preferred_element_type=jnp.float32)
            for k in range(NH))
    _split_outs(z, z0_ref, z1_ref, z2_ref, esedT_ref, act_ref[...])


def _final_body(a0_ref, a1_ref, a2_ref, ad_ref, fcw3_ref, fcb_ref, out_ref):
    hks = _elu_heads((a0_ref, a1_ref, a2_ref), ad_ref)
    logits = sum(jnp.dot(hks[k], fcw3_ref[k], preferred_element_type=jnp.float32)
                 for k in range(NH))
    out_ref[...] = logits + fcb_ref[...]


_ZOUT = (
    [jax.ShapeDtypeStruct((N, HF), jnp.float32)] * NH
    + [jax.ShapeDtypeStruct((SED, N), jnp.float32)]
)
_ZSPECS = (
    [pl.BlockSpec((BN, HF), lambda i: (i, 0))] * NH
    + [pl.BlockSpec((SED, BN), lambda i: (0, i))]
)
_ACCSPEC = pl.BlockSpec((NCORES, BN, HF), lambda i: (0, i, 0))
_ADSPEC = pl.BlockSpec((NH, NCORES, BN, 16), lambda i: (0, 0, i, 0))


def _tc_layer1(x, wc, act):
    return pl.pallas_call(
        _l1_body,
        grid=(GRID,),
        in_specs=[
            pl.BlockSpec((BN, 128), lambda i: (i, 0)),
            pl.BlockSpec((128, DH), lambda i: (0, 0)),
            pl.BlockSpec((SED, DH), lambda i: (0, 0)),
        ],
        out_specs=_ZSPECS,
        out_shape=_ZOUT,
    )(x, wc, act)


def _tc_layermid(a0, a1, a2, ad, wc3, act):
    return pl.pallas_call(
        _lmid_body,
        grid=(GRID,),
        in_specs=[
            _ACCSPEC, _ACCSPEC, _ACCSPEC, _ADSPEC,
            pl.BlockSpec((NH, HF, DH), lambda i: (0, 0, 0)),
            pl.BlockSpec((SED, DH), lambda i: (0, 0)),
        ],
        out_specs=_ZSPECS,
        out_shape=_ZOUT,
    )(a0, a1, a2, ad, wc3, act)


def _tc_final(a0, a1, a2, ad, fcw3, fcb):
    return pl.pallas_call(
        _final_body,
        grid=(GRID,),
        in_specs=[
            _ACCSPEC, _ACCSPEC, _ACCSPEC, _ADSPEC,
            pl.BlockSpec((NH, HF, NCLS), lambda i: (0, 0, 0)),
            pl.BlockSpec((1, NCLS), lambda i: (0, 0)),
        ],
        out_specs=pl.BlockSpec((BN, NCLS), lambda i: (i, 0)),
        out_shape=jax.ShapeDtypeStruct((N, NCLS), jnp.float32),
    )(a0, a1, a2, ad, fcw3, fcb)


_SC_OUT = (
    tuple(jax.ShapeDtypeStruct((NCORES * N, HF), jnp.float32)
          for _ in range(NH))
    + (jax.ShapeDtypeStruct((NH, NCORES * N, 16), jnp.float32),)
)

_SC_SCRATCH = [
    pltpu.VMEM((N,), jnp.float32),
    pltpu.VMEM((N,), jnp.float32),
    pltpu.VMEM((NCHUNK, CH), jnp.int32),
    pltpu.VMEM((NCHUNK, CH), jnp.int32),
    pltpu.VMEM((NB, CH, HF), jnp.float32),
    pltpu.VMEM((NB, CH, 16), jnp.float32),
    pltpu.VMEM((NB * CH,), jnp.float32),
    pltpu.VMEM((16,), jnp.float32),
    pltpu.VMEM_SHARED((N, HF), jnp.float32),
    pltpu.VMEM_SHARED((N, 16), jnp.float32),
] + [pltpu.SemaphoreType.DMA] * (3 * NB)


def _sc_body(src2_hbm, dst2_hbm, z0_hbm, z1_hbm, z2_hbm, esedT_hbm,
             zr_hbm, zd_hbm,
             o0_hbm, o1_hbm, o2_hbm, od_hbm,
             es_v, ed_v, srcv, dstv, rowsv, exrow, exv, mbuf,
             rowacc, denacc, *sems):
    gsems = sems[:NB]
    ssems = sems[NB:2 * NB]
    dsems = sems[2 * NB:]
    cid = lax.axis_index("c")
    sid = lax.axis_index("s")
    zvec = jnp.zeros((16,), jnp.float32)
    lane0 = jnp.where(lax.iota(jnp.int32, 16) == 0, 1.0, 0.0)

    def _ze(q, _):
        exrow[q // CH, q % CH, pl.ds(0, 16)] = zvec
        return 0
    lax.fori_loop(0, NB * CH, _ze, 0)

    trow0 = (cid * NSUB + sid) * NCHUNK
    pltpu.sync_copy(src2_hbm.at[pl.ds(trow0, NCHUNK)], srcv)
    pltpu.sync_copy(dst2_hbm.at[pl.ds(trow0, NCHUNK)], dstv)

    z_hbms = (z0_hbm, z1_hbm, z2_hbm)
    o_hbms = (o0_hbm, o1_hbm, o2_hbm)
    r0 = sid * NPT
    for h in range(NH):
        pltpu.sync_copy(zr_hbm.at[pl.ds(r0, NPT)], rowacc.at[pl.ds(r0, NPT)])
        pltpu.sync_copy(zd_hbm.at[pl.ds(r0, NPT)], denacc.at[pl.ds(r0, NPT)])
        pltpu.sync_copy(esedT_hbm.at[8 * h], es_v)
        pltpu.sync_copy(esedT_hbm.at[8 * (NH + h)], ed_v)
        plsc.subcore_barrier()

        def _mx(i, m):
            return jnp.maximum(m, es_v[pl.ds(i * 16, 16)])
        mvec = lax.fori_loop(0, N // 16, _mx,
                             jnp.full((16,), -1e30, jnp.float32))
        mbuf[pl.ds(0, 16)] = mvec
        maxes = plsc.load_gather(mbuf, [jnp.zeros((16,), jnp.int32)])
        for i in range(1, 16):
            maxes = jnp.maximum(
                maxes,
                plsc.load_gather(mbuf, [jnp.full((16,), i, jnp.int32)]))

        for b in range(NB):
            pltpu.async_copy(
                z_hbms[h].at[srcv.at[b]], rowsv.at[b], gsems[b])

        def _group(g, _):
            for b in range(NB):
                k = g * NB + b

                def _sc16(i, _, k=k, b=b):
                    si = srcv[k, pl.ds(i * 16, 16)]
                    di = dstv[k, pl.ds(i * 16, 16)]
                    esg = plsc.load_gather(es_v, [si])
                    edg = plsc.load_gather(ed_v, [di])
                    e = esg + edg
                    e = jnp.where(e >= 0.0, e, 0.01 * e)
                    m0 = edg + maxes
                    m0 = jnp.where(m0 >= 0.0, m0, 0.01 * m0)
                    ex = jnp.exp(jnp.maximum(e - m0, -75.0))
                    exv[pl.ds(b * CH + i * 16, 16)] = ex
                    ridx = lax.iota(jnp.int32, 16) + i * 16
                    plsc.store_scatter(
                        exrow, [jnp.full((16,), b, jnp.int32), ridx,
                                jnp.zeros((16,), jnp.int32)], ex)
                    return 0
                lax.fori_loop(0, CH // 16, _sc16, 0)
            scps = []
            for b in range(NB):
                k = g * NB + b
                pltpu.make_async_copy(
                    z_hbms[h].at[srcv.at[k]], rowsv.at[b], gsems[b]).wait()

                def _row(r4, _, b=b):
                    for u in range(4):
                        r = r4 * 4 + u
                        exs = plsc.load_gather(
                            exv, [jnp.full((16,), b * CH, jnp.int32) + r])
                        for j in range(HF // 16):
                            rowsv[b, r, pl.ds(j * 16, 16)] = (
                                rowsv[b, r, pl.ds(j * 16, 16)] * exs)
                    return 0
                lax.fori_loop(0, CH // 4, _row, 0)
                scps.append(pltpu.async_copy(
                    rowsv.at[b], rowacc.at[dstv.at[k]], ssems[b], add=True))
                scps.append(pltpu.async_copy(
                    exrow.at[b], denacc.at[dstv.at[k]], dsems[b], add=True))
            for b in range(NB):
                scps[2 * b].wait()
                scps[2 * b + 1].wait()

                @pl.when(g + 1 < NCHUNK // NB)
                def _(b=b, g=g):
                    k = (g + 1) * NB + b
                    pltpu.async_copy(
                        z_hbms[h].at[srcv.at[k]], rowsv.at[b], gsems[b])
            return 0
        lax.fori_loop(0, NCHUNK // NB, _group, 0)
        plsc.subcore_barrier()

        pltpu.sync_copy(rowacc.at[pl.ds(r0, NPT)],
                        o_hbms[h].at[pl.ds(cid * N + r0, NPT)])
        pltpu.sync_copy(denacc.at[pl.ds(r0, NPT)],
                        od_hbm.at[h].at[pl.ds(cid * N + r0, NPT)])
        plsc.subcore_barrier()


@functools.cache
def _get_sc_agg():
    mesh = plsc.VectorSubcoreMesh(
        core_axis_name="c", subcore_axis_name="s",
        num_cores=NCORES, num_subcores=NSUB)
    return pl.kernel(
        _sc_body,
        out_type=_SC_OUT,
        mesh=mesh,
        compiler_params=pltpu.CompilerParams(
            needs_layout_passes=False, use_tc_tiling_on_sc=False),
        scratch_types=_SC_SCRATCH,
    )


def _sc_agg(*args):
    return _get_sc_agg()(*args)


def _make_act(A):
    act = jnp.zeros((SED, DH), jnp.float32)
    for h in range(NH):
        act = act.at[8 * h, h * HF:(h + 1) * HF].set(A[h, :HF, 0])
        act = act.at[8 * (NH + h), h * HF:(h + 1) * HF].set(A[h, HF:, 0])
    return act


def kernel(x, edge_index, W1, A1, Wrest, Arest, fc_w, fc_b):
    src = edge_index[0].reshape(E // CH, CH)
    dst = edge_index[1].reshape(E // CH, CH)

    wc1 = jnp.concatenate([W1[h] for h in range(NH)], axis=1)
    act1 = _make_act(A1)
    wc3s = [jnp.transpose(Wrest[l], (1, 0, 2)).reshape(NH, HF, DH)
            for l in range(NLAYERS - 1)]
    acts = [_make_act(Arest[l]) for l in range(NLAYERS - 1)]
    fcw3 = fc_w.reshape(NH, HF, NCLS)
    fcb = fc_b.reshape(1, NCLS)

    zr = jnp.zeros((N, HF), jnp.float32)
    zd = jnp.zeros((N, 16), jnp.float32)

    z0, z1, z2, esedT = _tc_layer1(x, wc1, act1)
    for l in range(NLAYERS - 1):
        a0, a1, a2, ad = _sc_agg(src, dst, z0, z1, z2, esedT, zr, zd)
        a0 = a0.reshape(NCORES, N, HF)
        a1 = a1.reshape(NCORES, N, HF)
        a2 = a2.reshape(NCORES, N, HF)
        ad = ad.reshape(NH, NCORES, N, 16)
        z0, z1, z2, esedT = _tc_layermid(a0, a1, a2, ad, wc3s[l], acts[l])
    a0, a1, a2, ad = _sc_agg(src, dst, z0, z1, z2, esedT, zr, zd)
    a0 = a0.reshape(NCORES, N, HF)
    a1 = a1.reshape(NCORES, N, HF)
    a2 = a2.reshape(NCORES, N, HF)
    ad = ad.reshape(NH, NCORES, N, 16)
    return _tc_final(a0, a1, a2, ad, fcw3, fcb)
